# baseline (speedup 1.0000x reference)
import jax
import jax.numpy as jnp
from jax import lax
from jax.experimental import pallas as pl
from jax.experimental.pallas import tpu as pltpu
from jax.experimental.pallas import tpu_sc as plsc

B, R, D = 128, 28, 2048
K = 7
REM = R - K
BB = 8
NSTEPS = B // BB

NC, NS, L = 2, 16, 16
NW = NC * NS
BPW = B // NW
NCHUNK = D // L


def _ranks(g):
    nsq = jnp.sum(g * g, axis=1, keepdims=True)
    lt = nsq.T < nsq
    eq = nsq.T == nsq
    ir = jax.lax.broadcasted_iota(jnp.int32, (R, R), 0)
    is_ = jax.lax.broadcasted_iota(jnp.int32, (R, R), 1)
    tie = eq & (is_ < ir)
    return jnp.sum((lt | tie).astype(jnp.int32), axis=1, keepdims=True)


def _tc_kernel(gpos_ref, gneg_ref, pos_ref, neg_ref, syn_ref, posm_ref):
    for b in range(BB):
        gp = gpos_ref[b]
        gn = gneg_ref[b]
        pos = pos_ref[b]
        neg = neg_ref[b]

        rp = _ranks(gp)
        rn = _ranks(gn)
        top_p = rp >= REM

        sel = ((rp == rn.T) & top_p).astype(jnp.float32)
        gathered = jnp.dot(sel, neg, preferred_element_type=jnp.float32,
                           precision=jax.lax.Precision.HIGHEST)
        syn_ref[b] = jnp.where(top_p, gathered, pos)

        mean_p = jnp.sum(jnp.where(top_p, 0.0, pos), axis=0,
                         keepdims=True) / REM
        posm_ref[b] = jnp.where(top_p, mean_p, pos)


def _argmax_kernel(s_ref, cap_ref, imgn_ref):
    s = s_ref[...]
    ir = jax.lax.broadcasted_iota(jnp.int32, (B, B), 0)
    ic = jax.lax.broadcasted_iota(jnp.int32, (B, B), 1)
    s2 = jnp.where(ir == ic, s - 10.0, s)
    m1 = jnp.max(s2, axis=1, keepdims=True)
    cap_ref[...] = jnp.min(jnp.where(s2 == m1, ic, B), axis=1, keepdims=True)
    m0 = jnp.max(s2, axis=0, keepdims=True)
    imgn_ref[...] = jnp.min(jnp.where(s2 == m0, ir, B), axis=0, keepdims=True)


def _sc_negm_kernel(grad_hbm, neg_hbm, out_hbm, g_v, neg_v):
    wid = lax.axis_index("s") * NC + lax.axis_index("c")
    lanes = lax.iota(jnp.int32, L)
    inv = jnp.float32(1.0 / REM)
    one = jnp.ones((L,), jnp.float32)
    zero = jnp.zeros((L,), jnp.float32)

    def lanesum(x):
        for sh in (8, 4, 2, 1):
            idx = jnp.bitwise_xor(lanes, sh)
            x = x + x.at[idx].get(mode="promise_in_bounds",
                                  unique_indices=True)
        return x

    for j in range(BPW):
        b = wid * BPW + j
        pltpu.sync_copy(grad_hbm.at[B + b], g_v)
        pltpu.sync_copy(neg_hbm.at[b], neg_v)

        def nbody(i, accs):
            off = i * L
            out = []
            for r in range(R):
                v = g_v[pl.ds(r * D + off, L)]
                out.append(accs[r] + v * v)
            return tuple(out)
        accs = lax.fori_loop(0, NCHUNK, nbody,
                             tuple(zero for _ in range(R)))
        nv = [lanesum(accs[r]) for r in range(R)]

        c0 = zero
        for r in range(L):
            c0 = jnp.where(lanes == r, nv[r], c0)
        c1 = jnp.full((L,), 3.0e38, jnp.float32)
        for r in range(L, R):
            c1 = jnp.where(lanes == r - L, nv[r], c1)

        ws = []
        for r in range(R):
            m0 = (c0 < nv[r]) | ((c0 == nv[r]) & (lanes < r))
            m1 = (c1 < nv[r]) | ((c1 == nv[r]) & ((lanes + L) < r))
            pc = lanesum(jnp.where(m0, one, zero) + jnp.where(m1, one, zero))
            ws.append(jnp.where(pc >= float(REM), zero, one))
        ws = tuple(ws)

        def mobody(i, ws):
            off = i * L
            vs = [neg_v[pl.ds(r * D + off, L)] for r in range(R)]
            acc = zero
            for r in range(R):
                acc = acc + vs[r] * ws[r]
            m = acc * inv
            for r in range(R):
                neg_v[pl.ds(r * D + off, L)] = jnp.where(ws[r] > 0.5,
                                                         vs[r], m)
            return ws
        lax.fori_loop(0, NCHUNK, mobody, ws)

        pltpu.sync_copy(neg_v, out_hbm.at[b])


def _sc_negm(img_grad, img_neg):
    mesh = plsc.VectorSubcoreMesh(core_axis_name="c", subcore_axis_name="s")
    fn = pl.kernel(
        _sc_negm_kernel,
        mesh=mesh,
        out_type=jax.ShapeDtypeStruct((B, R * D), jnp.float32),
        scratch_types=[
            pltpu.VMEM((R * D,), jnp.float32),
            pltpu.VMEM((R * D,), jnp.float32),
        ],
    )
    out = fn(img_grad.reshape(2 * B, R * D), img_neg.reshape(B, R * D))
    return out.reshape(B, R, D)


def kernel(img_pos, img_neg, img_grad, scores):
    negm = _sc_negm(img_grad, img_neg)

    blk = pl.BlockSpec((BB, R, D), lambda i: (i, 0, 0))
    gblk = pl.BlockSpec((BB, R, D), lambda i: (i, 0, 0))
    gblk2 = pl.BlockSpec((BB, R, D), lambda i: (i + NSTEPS, 0, 0))
    syn, posm = pl.pallas_call(
        _tc_kernel,
        grid=(NSTEPS,),
        in_specs=[gblk, gblk2, blk, blk],
        out_specs=[blk, blk],
        out_shape=[jax.ShapeDtypeStruct((B, R, D), jnp.float32)] * 2,
    )(img_grad, img_grad, img_pos, img_neg)

    cap, imgn = pl.pallas_call(
        _argmax_kernel,
        out_shape=[jax.ShapeDtypeStruct((B, 1), jnp.int32),
                   jax.ShapeDtypeStruct((1, B), jnp.int32)],
    )(scores)
    return syn, posm, negm, cap.reshape(B), imgn.reshape(B)


# speedup vs baseline: 1.0550x; 1.0550x over previous
import jax
import jax.numpy as jnp
from jax import lax
from jax.experimental import pallas as pl
from jax.experimental.pallas import tpu as pltpu
from jax.experimental.pallas import tpu_sc as plsc

B, R, D = 128, 28, 2048
K = 7
REM = R - K
BB = 8
NSTEPS = B // BB

NC, NS, L = 2, 16, 16
NW = NC * NS
BPW = B // NW
NCHUNK = D // L


def _ranks(g):
    nsq = jnp.sum(g * g, axis=1, keepdims=True)
    lt = nsq.T < nsq
    eq = nsq.T == nsq
    ir = jax.lax.broadcasted_iota(jnp.int32, (R, R), 0)
    is_ = jax.lax.broadcasted_iota(jnp.int32, (R, R), 1)
    tie = eq & (is_ < ir)
    return jnp.sum((lt | tie).astype(jnp.int32), axis=1, keepdims=True)


def _tc_kernel(gpos_ref, gneg_ref, pos_ref, neg_ref, syn_ref, posm_ref):
    for b in range(BB):
        gp = gpos_ref[b]
        gn = gneg_ref[b]
        pos = pos_ref[b]
        neg = neg_ref[b]

        rp = _ranks(gp)
        rn = _ranks(gn)
        top_p = rp >= REM

        sel = ((rp == rn.T) & top_p).astype(jnp.float32)
        gathered = jnp.dot(sel, neg, preferred_element_type=jnp.float32,
                           precision=jax.lax.Precision.HIGHEST)
        syn_ref[b] = jnp.where(top_p, gathered, pos)

        mean_p = jnp.sum(jnp.where(top_p, 0.0, pos), axis=0,
                         keepdims=True) / REM
        posm_ref[b] = jnp.where(top_p, mean_p, pos)


def _argmax_kernel(s_ref, cap_ref, imgn_ref):
    s = s_ref[...]
    ir = jax.lax.broadcasted_iota(jnp.int32, (B, B), 0)
    ic = jax.lax.broadcasted_iota(jnp.int32, (B, B), 1)
    s2 = jnp.where(ir == ic, s - 10.0, s)
    m1 = jnp.max(s2, axis=1, keepdims=True)
    cap_ref[...] = jnp.min(jnp.where(s2 == m1, ic, B), axis=1, keepdims=True)
    m0 = jnp.max(s2, axis=0, keepdims=True)
    imgn_ref[...] = jnp.min(jnp.where(s2 == m0, ir, B), axis=0, keepdims=True)


def _sc_negm_kernel(grad_hbm, neg_hbm, out_hbm, g_v, neg_v):
    wid = lax.axis_index("s") * NC + lax.axis_index("c")
    lanes = lax.iota(jnp.int32, L)
    inv = jnp.float32(1.0 / REM)
    one = jnp.ones((L,), jnp.float32)
    zero = jnp.zeros((L,), jnp.float32)

    def lanesum(x):
        for sh in (8, 4, 2, 1):
            idx = jnp.bitwise_xor(lanes, sh)
            x = x + x.at[idx].get(mode="promise_in_bounds",
                                  unique_indices=True)
        return x

    for j in range(BPW):
        b = wid * BPW + j
        pltpu.sync_copy(grad_hbm.at[b], g_v)
        pltpu.sync_copy(neg_hbm.at[b], neg_v)

        def nbody(i, accs):
            off = i * L
            out = []
            for r in range(R):
                v = g_v[pl.ds(r * D + off, L)]
                out.append(accs[r] + v * v)
            return tuple(out)
        accs = lax.fori_loop(0, NCHUNK, nbody,
                             tuple(zero for _ in range(R)))
        nv = [lanesum(accs[r]) for r in range(R)]

        c0 = zero
        for r in range(L):
            c0 = jnp.where(lanes == r, nv[r], c0)
        c1 = jnp.full((L,), 3.0e38, jnp.float32)
        for r in range(L, R):
            c1 = jnp.where(lanes == r - L, nv[r], c1)

        ws = []
        for r in range(R):
            m0 = (c0 < nv[r]) | ((c0 == nv[r]) & (lanes < r))
            m1 = (c1 < nv[r]) | ((c1 == nv[r]) & ((lanes + L) < r))
            pc = lanesum(jnp.where(m0, one, zero) + jnp.where(m1, one, zero))
            ws.append(jnp.where(pc >= float(REM), zero, one))
        ws = tuple(ws)

        def mobody(i, ws):
            off = i * L
            vs = [neg_v[pl.ds(r * D + off, L)] for r in range(R)]
            acc = zero
            for r in range(R):
                acc = acc + vs[r] * ws[r]
            m = acc * inv
            for r in range(R):
                neg_v[pl.ds(r * D + off, L)] = jnp.where(ws[r] > 0.5,
                                                         vs[r], m)
            return ws
        lax.fori_loop(0, NCHUNK, mobody, ws)

        pltpu.sync_copy(neg_v, out_hbm.at[b])


def _sc_negm(img_grad, img_neg):
    mesh = plsc.VectorSubcoreMesh(core_axis_name="c", subcore_axis_name="s")
    fn = pl.kernel(
        _sc_negm_kernel,
        mesh=mesh,
        out_type=jax.ShapeDtypeStruct((B, R * D), jnp.float32),
        scratch_types=[
            pltpu.VMEM((R * D,), jnp.float32),
            pltpu.VMEM((R * D,), jnp.float32),
        ],
    )
    out = fn(img_grad[B:].reshape(B, R * D), img_neg.reshape(B, R * D))
    return out.reshape(B, R, D)


def kernel(img_pos, img_neg, img_grad, scores):
    negm = _sc_negm(img_grad, img_neg)

    blk = pl.BlockSpec((BB, R, D), lambda i: (i, 0, 0))
    gblk = pl.BlockSpec((BB, R, D), lambda i: (i, 0, 0))
    gblk2 = pl.BlockSpec((BB, R, D), lambda i: (i + NSTEPS, 0, 0))
    syn, posm = pl.pallas_call(
        _tc_kernel,
        grid=(NSTEPS,),
        in_specs=[gblk, gblk2, blk, blk],
        out_specs=[blk, blk],
        out_shape=[jax.ShapeDtypeStruct((B, R, D), jnp.float32)] * 2,
    )(img_grad, img_grad, img_pos, img_neg)

    cap, imgn = pl.pallas_call(
        _argmax_kernel,
        out_shape=[jax.ShapeDtypeStruct((B, 1), jnp.int32),
                   jax.ShapeDtypeStruct((1, B), jnp.int32)],
    )(scores)
    return syn, posm, negm, cap.reshape(B), imgn.reshape(B)
